# baseline (speedup 1.0000x reference)
import functools

import jax
import jax.numpy as jnp
from jax import lax
from jax.experimental import pallas as pl
from jax.experimental.pallas import tpu as pltpu
from jax.experimental.pallas import tpu_sc as plsc

_DIM = 32
_K = 8192
_N = 4608
_BN = 1152
_NGRID = _N // _BN
_BB = _BN // 576
_NB = 8

_RPW = _N // 32
_C = 72

_INV_COUNT = 1.0 / float(_N * _DIM)


def _tc_argmin_body(x_ref, e_ref, idx_ref, idxl_ref, dsum_ref):
    pid = pl.program_id(0)
    x = x_ref[...].reshape(_BN, _DIM)
    x2 = x + x
    xnorm = jnp.sum(x * x, axis=1, keepdims=True)
    e = e_ref[...]
    scores2 = jnp.dot(x2, e, preferred_element_type=jnp.float32)
    enorm = jnp.sum(e * e, axis=0, keepdims=True)
    ng = _K // 128
    m = (xnorm - scores2[:, :128]) + enorm[:, :128]
    gi = jnp.zeros((_BN, 128), jnp.float32)
    for g in range(1, ng):
        dg = (xnorm - scores2[:, g * 128:(g + 1) * 128]) + enorm[:, g * 128:(g + 1) * 128]
        lt = dg < m
        m = jnp.minimum(m, dg)
        gi = jnp.where(lt, jnp.float32(g), gi)
    gmin = jnp.min(m, axis=1)
    lidx = lax.broadcasted_iota(jnp.int32, (_BN, 128), 1).astype(jnp.float32)
    fidx = gi * 128.0 + lidx
    fbest = jnp.min(jnp.where(m == gmin[:, None], fidx, jnp.inf), axis=1)
    ivec = fbest.astype(jnp.int32)
    ibest = ivec.reshape(_BB, 576)
    for p in range(_NGRID):
        @pl.when(pid == p)
        def _():
            idx_ref[p * _BB:(p + 1) * _BB, :] = ibest
    idxl_ref[pl.ds(pid * _BN, _BN)] = ivec
    bsum = jnp.sum(gmin)[None, None] * jnp.float32(_INV_COUNT)

    @pl.when(pid == 0)
    def _():
        dsum_ref[...] = jnp.zeros((1, 1), jnp.float32)

    dsum_ref[...] += bsum


def _argmin_codes(flat, embed):
    return pl.pallas_call(
        _tc_argmin_body,
        grid=(_NGRID,),
        in_specs=[
            pl.BlockSpec((_BB, 576, _DIM), lambda i: (i, 0, 0)),
            pl.BlockSpec((_DIM, _K), lambda i: (0, 0)),
        ],
        out_specs=[
            pl.BlockSpec((_NB, 576), lambda i: (0, 0)),
            pl.BlockSpec((_N,), lambda i: (0,)),
            pl.BlockSpec((1, 1), lambda i: (0, 0)),
        ],
        out_shape=[
            jax.ShapeDtypeStruct((_NB, 576), jnp.int32),
            jax.ShapeDtypeStruct((_N,), jnp.int32),
            jax.ShapeDtypeStruct((1, 1), jnp.float32),
        ],
    )(flat, embed)


@functools.cache
def _sc_gather():
    @functools.partial(
        pl.kernel,
        mesh=plsc.VectorSubcoreMesh(core_axis_name="c", subcore_axis_name="s"),
        out_type=jax.ShapeDtypeStruct((_NB, 576, _DIM), jnp.float32),
        scratch_types=[
            pltpu.VMEM((_C,), jnp.int32),
            pltpu.VMEM((_C,), jnp.int32),
            pltpu.VMEM((_C, _DIM), jnp.float32),
            pltpu.VMEM((_C, _DIM), jnp.float32),
            pltpu.SemaphoreType.DMA,
        ],
        compiler_params=pltpu.CompilerParams(use_tc_tiling_on_sc=False),
    )
    def sc_body(emb_t, idx, q_out, idx_v0, idx_v1, rows_v0, rows_v1, sem):
        wid = lax.axis_index("s") * 2 + lax.axis_index("c")
        b = wid // 4
        off = (wid % 4) * _RPW
        flat = b * 576 + off
        pltpu.sync_copy(idx.at[pl.ds(flat, _C)], idx_v0)
        pltpu.sync_copy(idx.at[pl.ds(flat + _C, _C)], idx_v1)
        g0 = pltpu.async_copy(emb_t.at[idx_v0], rows_v0, sem)
        g1 = pltpu.async_copy(emb_t.at[idx_v1], rows_v1, sem)
        g0.wait()
        pltpu.sync_copy(rows_v0, q_out.at[b, pl.ds(off, _C)])
        g1.wait()
        pltpu.sync_copy(rows_v1, q_out.at[b, pl.ds(off + _C, _C)])

    return sc_body


def kernel(input, embed):
    idx, idx_lin, dsum = _argmin_codes(input, embed)
    emb_t = embed.T
    quantize = _sc_gather()(emb_t, idx_lin)
    embed_ind = idx
    diff = dsum.reshape(())
    return (quantize, embed_ind, diff, diff)


# speedup vs baseline: 1.0330x; 1.0330x over previous
import functools

import jax
import jax.numpy as jnp
from jax import lax
from jax.experimental import pallas as pl
from jax.experimental.pallas import tpu as pltpu
from jax.experimental.pallas import tpu_sc as plsc

_DIM = 32
_K = 8192
_N = 4608
_BN = 2304
_NGRID = _N // _BN
_BB = _BN // 576
_NB = 8

_RPW = _N // 32
_C = 72

_INV_COUNT = 1.0 / float(_N * _DIM)


def _tc_argmin_body(x_ref, e_ref, idx_ref, idxl_ref, dsum_ref):
    pid = pl.program_id(0)
    x = x_ref[...].reshape(_BN, _DIM)
    x2 = x + x
    xnorm = jnp.sum(x * x, axis=1, keepdims=True)
    e = e_ref[...]
    enorm = jnp.sum(e * e, axis=0, keepdims=True)
    kh = _K // 2
    ngh = kh // 128
    m = None
    gi = None
    for h in range(2):
        scores2 = jnp.dot(x2, e[:, h * kh:(h + 1) * kh],
                          preferred_element_type=jnp.float32)
        for gl in range(ngh):
            g = h * ngh + gl
            dg = ((xnorm - scores2[:, gl * 128:(gl + 1) * 128])
                  + enorm[:, g * 128:(g + 1) * 128])
            if m is None:
                m, gi = dg, jnp.zeros((_BN, 128), jnp.float32)
            else:
                lt = dg < m
                m = jnp.minimum(m, dg)
                gi = jnp.where(lt, jnp.float32(g), gi)
    gmin = jnp.min(m, axis=1)
    lidx = lax.broadcasted_iota(jnp.int32, (_BN, 128), 1).astype(jnp.float32)
    fidx = gi * 128.0 + lidx
    fbest = jnp.min(jnp.where(m == gmin[:, None], fidx, jnp.inf), axis=1)
    ivec = fbest.astype(jnp.int32)
    ibest = ivec.reshape(_BB, 576)
    for p in range(_NGRID):
        @pl.when(pid == p)
        def _():
            idx_ref[p * _BB:(p + 1) * _BB, :] = ibest
    idxl_ref[pl.ds(pid * _BN, _BN)] = ivec
    bsum = jnp.sum(gmin)[None, None] * jnp.float32(_INV_COUNT)

    @pl.when(pid == 0)
    def _():
        dsum_ref[...] = jnp.zeros((1, 1), jnp.float32)

    dsum_ref[...] += bsum


def _argmin_codes(flat, embed):
    return pl.pallas_call(
        _tc_argmin_body,
        grid=(_NGRID,),
        in_specs=[
            pl.BlockSpec((_BB, 576, _DIM), lambda i: (i, 0, 0)),
            pl.BlockSpec((_DIM, _K), lambda i: (0, 0)),
        ],
        out_specs=[
            pl.BlockSpec((_NB, 576), lambda i: (0, 0)),
            pl.BlockSpec((_N,), lambda i: (0,)),
            pl.BlockSpec((1, 1), lambda i: (0, 0)),
        ],
        out_shape=[
            jax.ShapeDtypeStruct((_NB, 576), jnp.int32),
            jax.ShapeDtypeStruct((_N,), jnp.int32),
            jax.ShapeDtypeStruct((1, 1), jnp.float32),
        ],
    )(flat, embed)


@functools.cache
def _sc_gather():
    @functools.partial(
        pl.kernel,
        mesh=plsc.VectorSubcoreMesh(core_axis_name="c", subcore_axis_name="s"),
        out_type=jax.ShapeDtypeStruct((_NB, 576, _DIM), jnp.float32),
        scratch_types=[
            pltpu.VMEM((_C,), jnp.int32),
            pltpu.VMEM((_C,), jnp.int32),
            pltpu.VMEM((_C, _DIM), jnp.float32),
            pltpu.VMEM((_C, _DIM), jnp.float32),
            pltpu.SemaphoreType.DMA,
        ],
        compiler_params=pltpu.CompilerParams(use_tc_tiling_on_sc=False),
    )
    def sc_body(emb_t, idx, q_out, idx_v0, idx_v1, rows_v0, rows_v1, sem):
        wid = lax.axis_index("s") * 2 + lax.axis_index("c")
        b = wid // 4
        off = (wid % 4) * _RPW
        flat = b * 576 + off
        pltpu.sync_copy(idx.at[pl.ds(flat, _C)], idx_v0)
        pltpu.sync_copy(idx.at[pl.ds(flat + _C, _C)], idx_v1)
        g0 = pltpu.async_copy(emb_t.at[idx_v0], rows_v0, sem)
        g1 = pltpu.async_copy(emb_t.at[idx_v1], rows_v1, sem)
        g0.wait()
        pltpu.sync_copy(rows_v0, q_out.at[b, pl.ds(off, _C)])
        g1.wait()
        pltpu.sync_copy(rows_v1, q_out.at[b, pl.ds(off + _C, _C)])

    return sc_body


def kernel(input, embed):
    idx, idx_lin, dsum = _argmin_codes(input, embed)
    emb_t = embed.T
    quantize = _sc_gather()(emb_t, idx_lin)
    embed_ind = idx
    diff = dsum.reshape(())
    return (quantize, embed_ind, diff, diff)
